# baseline (speedup 1.0000x reference)
import functools

import jax
import jax.numpy as jnp
from jax import lax
from jax.experimental import pallas as pl
from jax.experimental.pallas import tpu as pltpu
from jax.experimental.pallas import tpu_sc as plsc

PAD_ID = 0
START_ID = 0
LANES = 16
NBUF = 3
SUB = 32


@functools.partial(jax.jit, static_argnames=("B", "S", "D"))
def _embed_lookup(ids2d, wte, wpe, ehs, B, S, D):
    NC, NS = 2, 16
    NW = NC * NS
    CH = S // NW
    nsub = B * (CH // SUB)
    ECH = (B * S) // NW

    mesh = plsc.VectorSubcoreMesh(core_axis_name="c", subcore_axis_name="s")

    @functools.partial(
        pl.kernel,
        mesh=mesh,
        out_type=(
            jax.ShapeDtypeStruct((B * S, D), jnp.float32),
            jax.ShapeDtypeStruct((B * S, D), jnp.float32),
        ),
        scratch_types=[
            pltpu.VMEM((B, CH), jnp.int32),
            pltpu.VMEM((CH, D), jnp.float32),
        ]
        + [pltpu.VMEM((SUB, D), jnp.float32) for _ in range(NBUF)]
        + [pltpu.SemaphoreType.DMA for _ in range(2 * NBUF + 1)],
    )
    def k(ids_hbm, wte_hbm, wpe_hbm, ehs_hbm, out_hbm, enc_hbm, idx_v, wpe_v, *bufs_sems):
        rows = bufs_sems[:NBUF]
        gsem = bufs_sems[NBUF : 2 * NBUF]
        wsem = bufs_sems[2 * NBUF : 3 * NBUF]
        esem = bufs_sems[3 * NBUF]
        wid = lax.axis_index("s") * NC + lax.axis_index("c")
        s0 = wid * CH

        e0 = wid * ECH
        enc_copy = pltpu.async_copy(
            ehs_hbm.at[pl.ds(e0, ECH), :], enc_hbm.at[pl.ds(e0, ECH), :], esem
        )

        for b in range(B):
            pltpu.sync_copy(ids_hbm.at[b, pl.ds(s0, CH)], idx_v.at[b])
        pltpu.sync_copy(wpe_hbm.at[pl.ds(s0, CH), :], wpe_v)

        writes = [None] * NBUF

        def start_gather(j):
            p = j % NBUF
            if writes[p] is not None:
                writes[p].wait()
            b, h = j // (CH // SUB), j % (CH // SUB)
            return pltpu.async_copy(
                wte_hbm.at[idx_v.at[b, pl.ds(h * SUB, SUB)]], rows[p], gsem[p]
            )

        def make_add(p, h):
            def add_row(i, _):
                for jj in range(D // LANES):
                    sl = pl.ds(jj * LANES, LANES)
                    plsc.addupdate(rows[p].at[i, sl], wpe_v[h * SUB + i, sl])
                return _

            return add_row

        gathers = [None] * NBUF
        gathers[0] = start_gather(0)
        for j in range(nsub):
            p = j % NBUF
            if j + 1 < nsub:
                gathers[(j + 1) % NBUF] = start_gather(j + 1)
            gathers[p].wait()
            b, h = j // (CH // SUB), j % (CH // SUB)
            lax.fori_loop(0, SUB, make_add(p, h), 0)
            writes[p] = pltpu.async_copy(
                rows[p], out_hbm.at[pl.ds(b * S + s0 + h * SUB, SUB), :], wsem[p]
            )
        for p in range(NBUF):
            if writes[p] is not None:
                writes[p].wait()
        enc_copy.wait()

    return k(ids2d, wte, wpe, ehs)


def kernel(encoder_hidden_states, labels, metadata, wte, wpe):
    B, S = labels.shape
    D = wte.shape[1]

    ids = jnp.concatenate(
        [jnp.full((B, 1), START_ID, labels.dtype), labels[:, :-1]], axis=1
    )
    ids = jnp.where(ids == -100, PAD_ID, ids)

    enc_b, enc_s, enc_d = encoder_hidden_states.shape
    token_emb, enc_out = _embed_lookup(
        ids, wte, wpe, encoder_hidden_states.reshape(enc_b * enc_s, enc_d), B, S, D
    )
    token_emb = token_emb.reshape(B, S, D)
    enc_out = enc_out.reshape(enc_b, enc_s, enc_d)

    encoder_extended_attention_mask = jnp.zeros(
        (enc_b, 1, 1, enc_s), dtype=jnp.float32
    )

    return (
        enc_out,
        token_emb,
        encoder_extended_attention_mask,
        metadata,
        ids,
        labels,
    )


# speedup vs baseline: 11.0231x; 11.0231x over previous
import functools

import jax
import jax.numpy as jnp
from jax import lax
from jax.experimental import pallas as pl
from jax.experimental.pallas import tpu as pltpu
from jax.experimental.pallas import tpu_sc as plsc

PAD_ID = 0
START_ID = 0
LANES = 16
SUB = 16


@functools.partial(jax.jit, static_argnames=("B", "S", "D"))
def _embed_lookup(ids2d, wte, wpe, B, S, D):
    NC, NS = 2, 16
    NW = NC * NS
    CH = S // NW
    nst = CH // SUB
    G = B * SUB

    mesh = plsc.VectorSubcoreMesh(core_axis_name="c", subcore_axis_name="s")

    @functools.partial(
        pl.kernel,
        mesh=mesh,
        out_type=jax.ShapeDtypeStruct((B * S, D), jnp.float32),
        scratch_types=[
            pltpu.VMEM((B, CH), jnp.int32),
            pltpu.VMEM((G,), jnp.int32),
            pltpu.VMEM((G,), jnp.int32),
            pltpu.VMEM((G, D), jnp.float32),
            pltpu.VMEM((G, D), jnp.float32),
            pltpu.VMEM((SUB, D), jnp.float32),
            pltpu.VMEM((SUB, D), jnp.float32),
            pltpu.SemaphoreType.DMA,
            pltpu.SemaphoreType.DMA,
            pltpu.SemaphoreType.DMA,
            pltpu.SemaphoreType.DMA,
            pltpu.SemaphoreType.DMA,
            pltpu.SemaphoreType.DMA,
        ],
    )
    def k(ids_hbm, wte_hbm, wpe_hbm, out_hbm, idx_v, l0, l1, r0, r1, w0, w1,
          g0, g1, p0, p1, s0_, s1_):
        lists, rows, wpeb = [l0, l1], [r0, r1], [w0, w1]
        gsem, psem, wsem = [g0, g1], [p0, p1], [s0_, s1_]
        wid = lax.axis_index("s") * NC + lax.axis_index("c")
        s0 = wid * CH

        for b in range(B):
            pltpu.sync_copy(ids_hbm.at[b, pl.ds(s0, CH)], idx_v.at[b])

        gathers = [None, None]
        wloads = [None, None]
        writes = [[], []]

        def issue(h):
            p = h % 2
            for wcopy in writes[p]:
                wcopy.wait()
            writes[p] = []
            for b in range(B):
                lists[p][pl.ds(b * SUB, SUB)] = idx_v[b, pl.ds(h * SUB, SUB)]
            gathers[p] = pltpu.async_copy(wte_hbm.at[lists[p]], rows[p], gsem[p])
            wloads[p] = pltpu.async_copy(
                wpe_hbm.at[pl.ds(s0 + h * SUB, SUB), :], wpeb[p], psem[p]
            )

        def make_add(p):
            def add_row(i, _):
                for jj in range(D // LANES):
                    sl = pl.ds(jj * LANES, LANES)
                    w = wpeb[p][i, sl]
                    for b in range(B):
                        plsc.addupdate(rows[p].at[b * SUB + i, sl], w)
                return _

            return add_row

        issue(0)
        for h in range(nst):
            p = h % 2
            if h + 1 < nst:
                issue(h + 1)
            gathers[p].wait()
            wloads[p].wait()
            lax.fori_loop(0, SUB, make_add(p), 0)
            writes[p] = [
                pltpu.async_copy(
                    rows[p].at[pl.ds(b * SUB, SUB), :],
                    out_hbm.at[pl.ds(b * S + s0 + h * SUB, SUB), :],
                    wsem[p],
                )
                for b in range(B)
            ]
        for p in range(2):
            for wcopy in writes[p]:
                wcopy.wait()

    return k(ids2d, wte, wpe)


def kernel(encoder_hidden_states, labels, metadata, wte, wpe):
    B, S = labels.shape
    D = wte.shape[1]

    ids = jnp.concatenate(
        [jnp.full((B, 1), START_ID, labels.dtype), labels[:, :-1]], axis=1
    )
    ids = jnp.where(ids == -100, PAD_ID, ids)

    token_emb = _embed_lookup(ids, wte, wpe, B, S, D)
    token_emb = token_emb.reshape(B, S, D)

    enc_b, enc_s, _ = encoder_hidden_states.shape
    encoder_extended_attention_mask = jnp.zeros(
        (enc_b, 1, 1, enc_s), dtype=jnp.float32
    )

    return (
        encoder_hidden_states,
        token_emb,
        encoder_extended_attention_mask,
        metadata,
        ids,
        labels,
    )
